# baseline (device time: 119057 ns/iter reference)
import jax
import jax.numpy as jnp
from jax import lax
from jax.experimental import pallas as pl
from jax.experimental.pallas import tpu as pltpu

N_DEV = 16
R_HOPS = 8
L_HOPS = 7


def kernel(t, W):
    m, k = t.shape
    _, n = W.shape
    chunk = m // N_DEV
    sub = chunk // 2

    def body(t_ref, w_ref, out_ref, send_buf, rs_recv_r, rs_recv_l,
             ag_own, ag_recv_r, ag_recv_l, t_stage, w_stage, out_stage,
             send_sems, rs_sems_r, rs_sems_l, ag_sems_r, ag_sems_l,
             t_sems, w_sem, out_sems):
        d = lax.axis_index("i")
        right = lax.rem(d + 1, N_DEV)
        left = lax.rem(d + N_DEV - 1, N_DEV)

        streams = [(0, True), (sub, True), (0, False), (sub, False)]
        n_hops = {True: R_HOPS, False: L_HOPS}
        ag_hops = {True: L_HOPS, False: R_HOPS}

        barrier_sem = pltpu.get_barrier_semaphore()
        for nbr in (left, right):
            pl.semaphore_signal(
                barrier_sem, inc=1,
                device_id=(nbr,), device_id_type=pl.DeviceIdType.MESH,
            )
        pl.semaphore_wait(barrier_sem, 2)

        def rs_send_chunk(s, rwd):
            if rwd:
                return lax.rem(d + R_HOPS - s, N_DEV)
            return lax.rem(d + N_DEV - L_HOPS + s, N_DEV)

        def t_dma(i, s, rwd):
            off = streams[i][0]
            c = rs_send_chunk(s, rwd)
            return pltpu.make_async_copy(
                t_ref.at[pl.ds(c * chunk + off, sub), :],
                t_stage.at[i, s % 2],
                t_sems.at[i, s % 2],
            )

        def rs_desc(i, s, slot, rwd):
            return pltpu.make_async_remote_copy(
                src_ref=send_buf.at[i, slot],
                dst_ref=(rs_recv_r if rwd else rs_recv_l).at[i % 2, s],
                send_sem=send_sems.at[i, slot],
                recv_sem=(rs_sems_r if rwd else rs_sems_l).at[i % 2, s],
                device_id=(right if rwd else left,),
                device_id_type=pl.DeviceIdType.MESH,
            )

        def ag_desc(i, h, rwd):
            buf = ag_recv_r if rwd else ag_recv_l
            src = ag_own.at[i % 2] if h == 0 else buf.at[i % 2, h - 1]
            return pltpu.make_async_remote_copy(
                src_ref=src,
                dst_ref=buf.at[i % 2, h],
                send_sem=send_sems.at[i, h % 2],
                recv_sem=(ag_sems_r if rwd else ag_sems_l).at[i % 2, h],
                device_id=(right if rwd else left,),
                device_id_type=pl.DeviceIdType.MESH,
            )

        def out_dma(j, slot, c):
            return pltpu.make_async_copy(
                out_stage.at[j, slot],
                out_ref.at[pl.ds(c * chunk + j * sub, sub), :],
                out_sems.at[j, slot],
            )

        pltpu.make_async_copy(w_ref, w_stage, w_sem).start()
        for i, (off, rwd) in enumerate(streams):
            t_dma(i, 0, rwd).start()
            t_dma(i, 1, rwd).start()

        rs_sent = [[] for _ in streams]
        ag_sent = [[] for _ in streams]
        for i, (off, rwd) in enumerate(streams):
            t_dma(i, 0, rwd).wait()
            send_buf[i, 0, :, :] = t_stage[i, 0].astype(jnp.bfloat16)
            dsc = rs_desc(i, 0, 0, rwd)
            dsc.start()
            rs_sent[i].append(dsc)

        for s in range(R_HOPS):
            for i, (off, rwd) in enumerate(streams):
                hops = n_hops[rwd]
                if s >= hops - 1:
                    continue
                if s >= 1:
                    rs_sent[i][s - 1].wait_send()
                t_dma(i, s + 1, rwd).wait()
                send_buf[i, (s + 1) % 2, :, :] = (
                    t_stage[i, (s + 1) % 2].astype(jnp.bfloat16)
                )
                if s + 2 <= (hops if rwd else hops - 1):
                    t_dma(i, s + 2, rwd).start()
            for i, (off, rwd) in enumerate(streams):
                hops = n_hops[rwd]
                if s >= hops:
                    continue
                recv_buf = rs_recv_r if rwd else rs_recv_l
                rs_desc(i, s, 0, rwd).wait_recv()
                if s < hops - 1:
                    slot = (s + 1) % 2
                    send_buf[i, slot, :, :] = (
                        send_buf[i, slot] + recv_buf[i % 2, s]
                    )
                    dsc = rs_desc(i, s + 1, slot, rwd)
                    dsc.start()
                    rs_sent[i].append(dsc)

        pltpu.make_async_copy(w_ref, w_stage, w_sem).wait()
        for j in range(2):
            off = j * sub
            t_dma(j, R_HOPS, True).wait()
            red = (
                rs_recv_r[j, R_HOPS - 1].astype(jnp.float32)
                + rs_recv_l[j, L_HOPS - 1].astype(jnp.float32)
                + t_stage[j, R_HOPS % 2]
            )
            res = jnp.dot(
                red, w_stage[:, :], preferred_element_type=jnp.float32
            )
            ag_own[j, :, :] = res.astype(jnp.bfloat16)
            for i, (ioff, rwd) in enumerate(streams):
                if ioff != off:
                    continue
                rs_sent[i][n_hops[rwd] - 2].wait_send()
                rs_sent[i][n_hops[rwd] - 1].wait_send()
                dsc = ag_desc(i, 0, rwd)
                dsc.start()
                ag_sent[i].append(dsc)
            out_stage[j, 0, :, :] = res
            out_dma(j, 0, d).start()

        for h in range(R_HOPS):
            for i, (off, rwd) in enumerate(streams):
                hops = ag_hops[rwd]
                if h >= hops:
                    continue
                j = i % 2
                if rwd:
                    rc = lax.rem(d + N_DEV - 1 - h, N_DEV)
                    slot = 1 + h
                else:
                    rc = lax.rem(d + 1 + h, N_DEV)
                    slot = 1 + L_HOPS + h
                ag_desc(i, h, rwd).wait_recv()
                if h < hops - 1:
                    if h >= 1:
                        ag_sent[i][h - 1].wait_send()
                    dsc = ag_desc(i, h + 1, rwd)
                    dsc.start()
                    ag_sent[i].append(dsc)
                buf = ag_recv_r if rwd else ag_recv_l
                out_stage[j, slot, :, :] = buf[j, h].astype(jnp.float32)
                out_dma(j, slot, rc).start()
        for i, (off, rwd) in enumerate(streams):
            hops = ag_hops[rwd]
            ag_sent[i][hops - 2].wait_send()
            ag_sent[i][hops - 1].wait_send()
        for j in range(2):
            for slot in range(N_DEV):
                c = d if slot == 0 else (
                    lax.rem(d + N_DEV - slot, N_DEV) if slot <= L_HOPS
                    else lax.rem(d + slot - L_HOPS, N_DEV)
                )
                out_dma(j, slot, c).wait()

    return pl.pallas_call(
        body,
        out_shape=jax.ShapeDtypeStruct((m, n), jnp.float32),
        in_specs=[
            pl.BlockSpec(memory_space=pl.ANY),
            pl.BlockSpec(memory_space=pl.ANY),
        ],
        out_specs=pl.BlockSpec(memory_space=pl.ANY),
        scratch_shapes=[
            pltpu.VMEM((4, 2, sub, k), jnp.bfloat16),
            pltpu.VMEM((2, R_HOPS, sub, k), jnp.bfloat16),
            pltpu.VMEM((2, L_HOPS, sub, k), jnp.bfloat16),
            pltpu.VMEM((2, sub, k), jnp.bfloat16),
            pltpu.VMEM((2, L_HOPS, sub, k), jnp.bfloat16),
            pltpu.VMEM((2, R_HOPS, sub, k), jnp.bfloat16),
            pltpu.VMEM((4, 2, sub, k), jnp.float32),
            pltpu.VMEM((k, n), jnp.float32),
            pltpu.VMEM((2, N_DEV, sub, k), jnp.float32),
            pltpu.SemaphoreType.DMA((4, 2)),
            pltpu.SemaphoreType.DMA((2, R_HOPS)),
            pltpu.SemaphoreType.DMA((2, L_HOPS)),
            pltpu.SemaphoreType.DMA((2, L_HOPS)),
            pltpu.SemaphoreType.DMA((2, R_HOPS)),
            pltpu.SemaphoreType.DMA((4, 2)),
            pltpu.SemaphoreType.DMA(()),
            pltpu.SemaphoreType.DMA((2, N_DEV)),
        ],
        compiler_params=pltpu.CompilerParams(
            collective_id=0, vmem_limit_bytes=100 * 1024 * 1024
        ),
    )(t, W)


# device time: 117288 ns/iter; 1.0151x vs baseline; 1.0151x over previous
import jax
import jax.numpy as jnp
from jax import lax
from jax.experimental import pallas as pl
from jax.experimental.pallas import tpu as pltpu

N_DEV = 16
R_HOPS = 8
L_HOPS = 7


def kernel(t, W):
    m, k = t.shape
    _, n = W.shape
    chunk = m // N_DEV
    sub = chunk // 2

    def body(t_ref, w_ref, out_ref, send_buf, rs_recv_r, rs_recv_l,
             ag_own, ag_recv_r, ag_recv_l, t_stage, w_stage, out_stage,
             send_sems, rs_sems_r, rs_sems_l, ag_sems_r, ag_sems_l,
             t_sems, w_sem, out_sems):
        d = lax.axis_index("i")
        right = lax.rem(d + 1, N_DEV)
        left = lax.rem(d + N_DEV - 1, N_DEV)

        streams = [(0, True), (sub, True), (0, False), (sub, False)]
        n_hops = {True: R_HOPS, False: L_HOPS}

        barrier_sem = pltpu.get_barrier_semaphore()
        for nbr in (left, right):
            pl.semaphore_signal(
                barrier_sem, inc=1,
                device_id=(nbr,), device_id_type=pl.DeviceIdType.MESH,
            )
        pl.semaphore_wait(barrier_sem, 2)

        def rs_send_chunk(s, rwd):
            if rwd:
                return lax.rem(d + R_HOPS - s, N_DEV)
            return lax.rem(d + N_DEV - L_HOPS + s, N_DEV)

        def t_dma(i, s, rwd):
            off = streams[i][0]
            c = rs_send_chunk(s, rwd)
            return pltpu.make_async_copy(
                t_ref.at[pl.ds(c * chunk + off, sub), :],
                t_stage.at[i, s % 2],
                t_sems.at[i, s % 2],
            )

        def rs_desc(i, s, slot, rwd):
            return pltpu.make_async_remote_copy(
                src_ref=send_buf.at[i, slot],
                dst_ref=(rs_recv_r if rwd else rs_recv_l).at[i % 2, s],
                send_sem=send_sems.at[i, slot],
                recv_sem=(rs_sems_r if rwd else rs_sems_l).at[i % 2, s],
                device_id=(right if rwd else left,),
                device_id_type=pl.DeviceIdType.MESH,
            )

        def ag_desc(i, h, rwd):
            buf = ag_recv_r if rwd else ag_recv_l
            src = ag_own.at[i % 2] if h == 0 else buf.at[i % 2, h - 1]
            return pltpu.make_async_remote_copy(
                src_ref=src,
                dst_ref=buf.at[i % 2, h],
                send_sem=send_sems.at[i, h % 2],
                recv_sem=(ag_sems_r if rwd else ag_sems_l).at[i % 2, h],
                device_id=(right if rwd else left,),
                device_id_type=pl.DeviceIdType.MESH,
            )

        def out_dma(j, slot, c):
            return pltpu.make_async_copy(
                out_stage.at[j, slot],
                out_ref.at[pl.ds(c * chunk + j * sub, sub), :],
                out_sems.at[j, slot],
            )

        pltpu.make_async_copy(w_ref, w_stage, w_sem).start()
        for i, (off, rwd) in enumerate(streams):
            t_dma(i, 0, rwd).start()
            t_dma(i, 1, rwd).start()

        rs_sent = [[] for _ in streams]
        ag_sent = [[] for _ in streams]
        for i, (off, rwd) in enumerate(streams):
            t_dma(i, 0, rwd).wait()
            send_buf[i, 0, :, :] = t_stage[i, 0].astype(jnp.bfloat16)
            dsc = rs_desc(i, 0, 0, rwd)
            dsc.start()
            rs_sent[i].append(dsc)

        for s in range(R_HOPS):
            for i, (off, rwd) in enumerate(streams):
                hops = n_hops[rwd]
                if s >= hops - 1:
                    continue
                if s >= 1:
                    rs_sent[i][s - 1].wait_send()
                t_dma(i, s + 1, rwd).wait()
                send_buf[i, (s + 1) % 2, :, :] = (
                    t_stage[i, (s + 1) % 2].astype(jnp.bfloat16)
                )
                if s + 2 <= (hops if rwd else hops - 1):
                    t_dma(i, s + 2, rwd).start()
            for i, (off, rwd) in enumerate(streams):
                hops = n_hops[rwd]
                if s >= hops:
                    continue
                recv_buf = rs_recv_r if rwd else rs_recv_l
                rs_desc(i, s, 0, rwd).wait_recv()
                if s < hops - 1:
                    slot = (s + 1) % 2
                    send_buf[i, slot, :, :] = (
                        send_buf[i, slot] + recv_buf[i % 2, s]
                    )
                    dsc = rs_desc(i, s + 1, slot, rwd)
                    dsc.start()
                    rs_sent[i].append(dsc)

        pltpu.make_async_copy(w_ref, w_stage, w_sem).wait()
        for j in range(2):
            off = j * sub
            t_dma(j, R_HOPS, True).wait()
            red = (
                rs_recv_r[j, R_HOPS - 1].astype(jnp.float32)
                + rs_recv_l[j, L_HOPS - 1].astype(jnp.float32)
                + t_stage[j, R_HOPS % 2]
            )
            res = jnp.dot(
                red, w_stage[:, :], preferred_element_type=jnp.float32
            )
            ag_own[j, :, :] = res.astype(jnp.bfloat16)
            for i, (ioff, rwd) in enumerate(streams):
                if ioff != off:
                    continue
                rs_sent[i][n_hops[rwd] - 2].wait_send()
                rs_sent[i][n_hops[rwd] - 1].wait_send()
                dsc = ag_desc(i, 0, rwd)
                dsc.start()
                ag_sent[i].append(dsc)
            out_stage[j, 0, :, :] = res
            out_dma(j, 0, d).start()

        for h in range(R_HOPS):
            for i, (off, rwd) in enumerate(streams):
                hops = n_hops[rwd]
                if h >= hops:
                    continue
                j = i % 2
                if rwd:
                    rc = lax.rem(d + N_DEV - 1 - h, N_DEV)
                    slot = 1 + h
                else:
                    rc = lax.rem(d + 1 + h, N_DEV)
                    slot = 1 + R_HOPS + h
                ag_desc(i, h, rwd).wait_recv()
                if h < hops - 1:
                    if h >= 1:
                        ag_sent[i][h - 1].wait_send()
                    dsc = ag_desc(i, h + 1, rwd)
                    dsc.start()
                    ag_sent[i].append(dsc)
                buf = ag_recv_r if rwd else ag_recv_l
                out_stage[j, slot, :, :] = buf[j, h].astype(jnp.float32)
                out_dma(j, slot, rc).start()
        for i, (off, rwd) in enumerate(streams):
            hops = n_hops[rwd]
            ag_sent[i][hops - 2].wait_send()
            ag_sent[i][hops - 1].wait_send()
        for j in range(2):
            for slot in range(N_DEV):
                c = d if slot == 0 else (
                    lax.rem(d + N_DEV - slot, N_DEV) if slot <= R_HOPS
                    else lax.rem(d + slot - R_HOPS, N_DEV)
                )
                out_dma(j, slot, c).wait()

    return pl.pallas_call(
        body,
        out_shape=jax.ShapeDtypeStruct((m, n), jnp.float32),
        in_specs=[
            pl.BlockSpec(memory_space=pl.ANY),
            pl.BlockSpec(memory_space=pl.ANY),
        ],
        out_specs=pl.BlockSpec(memory_space=pl.ANY),
        scratch_shapes=[
            pltpu.VMEM((4, 2, sub, k), jnp.bfloat16),
            pltpu.VMEM((2, R_HOPS, sub, k), jnp.bfloat16),
            pltpu.VMEM((2, L_HOPS, sub, k), jnp.bfloat16),
            pltpu.VMEM((2, sub, k), jnp.bfloat16),
            pltpu.VMEM((2, R_HOPS, sub, k), jnp.bfloat16),
            pltpu.VMEM((2, L_HOPS, sub, k), jnp.bfloat16),
            pltpu.VMEM((4, 2, sub, k), jnp.float32),
            pltpu.VMEM((k, n), jnp.float32),
            pltpu.VMEM((2, N_DEV, sub, k), jnp.float32),
            pltpu.SemaphoreType.DMA((4, 2)),
            pltpu.SemaphoreType.DMA((2, R_HOPS)),
            pltpu.SemaphoreType.DMA((2, L_HOPS)),
            pltpu.SemaphoreType.DMA((2, R_HOPS)),
            pltpu.SemaphoreType.DMA((2, L_HOPS)),
            pltpu.SemaphoreType.DMA((4, 2)),
            pltpu.SemaphoreType.DMA(()),
            pltpu.SemaphoreType.DMA((2, N_DEV)),
        ],
        compiler_params=pltpu.CompilerParams(
            collective_id=0, vmem_limit_bytes=100 * 1024 * 1024
        ),
    )(t, W)
